# baseline (speedup 1.0000x reference)
import functools

import jax
import jax.numpy as jnp
from jax import lax
from jax.experimental import pallas as pl
from jax.experimental.pallas import tpu as pltpu
from jax.experimental.pallas import tpu_sc as plsc

D = 128
NBUF = 4

_info = plsc.get_sparse_core_info()
NC, NS = _info.num_cores, _info.num_subcores
NW = NC * NS


@functools.lru_cache(maxsize=None)
def _make_gather(rows_per_w: int, seq: int):
    mesh = plsc.VectorSubcoreMesh(core_axis_name="c", subcore_axis_name="s")
    n_blocks = rows_per_w // NBUF
    nsub = 5
    sub = seq // nsub
    total = NW * rows_per_w * seq

    def body(ids_hbm, table_hbm, out_hbm, idx_v, *rest):
        bufs = rest[:NBUF]
        gsems = rest[NBUF:2 * NBUF]
        wsems = rest[2 * NBUF:3 * NBUF]
        wid = lax.axis_index("s") * NC + lax.axis_index("c")
        row0 = wid * (rows_per_w * seq)

        pltpu.sync_copy(ids_hbm.at[pl.ds(wid * rows_per_w, rows_per_w)], idx_v)

        def start_gather(r, b):
            for h in range(nsub):
                pltpu.async_copy(
                    table_hbm.at[idx_v.at[r, pl.ds(h * sub, sub)]],
                    bufs[b].at[pl.ds(h * sub, sub)], gsems[b])

        def wait_gather(r, b):
            for h in range(nsub):
                pltpu.make_async_copy(
                    table_hbm.at[idx_v.at[r, pl.ds(h * sub, sub)]],
                    bufs[b].at[pl.ds(h * sub, sub)], gsems[b]).wait()

        def start_write(r, b):
            pltpu.async_copy(bufs[b],
                             out_hbm.at[pl.ds(row0 + r * seq, seq)],
                             wsems[b])

        def wait_write(r, b):
            pltpu.make_async_copy(bufs[b],
                                  out_hbm.at[pl.ds(row0 + r * seq, seq)],
                                  wsems[b]).wait()

        for b in range(NBUF):
            start_gather(b, b)

        def block(i, carry):
            base = i * NBUF
            for b in range(NBUF):
                wait_gather(base + b, b)
                start_write(base + b, b)
            for b in range(NBUF):
                wait_write(base + b, b)
                start_gather(base + NBUF + b, b)
            return carry

        lax.fori_loop(0, n_blocks - 1, block, 0)

        base = (n_blocks - 1) * NBUF
        for b in range(NBUF):
            wait_gather(base + b, b)
            start_write(base + b, b)
        for b in range(NBUF):
            wait_write(base + b, b)

    return pl.kernel(
        body,
        out_type=jax.ShapeDtypeStruct((total, D), jnp.float32),
        mesh=mesh,
        compiler_params=pltpu.CompilerParams(use_tc_tiling_on_sc=False),
        scratch_types=(
            [pltpu.VMEM((rows_per_w, seq), jnp.int32)]
            + [pltpu.VMEM((seq, D), jnp.float32) for _ in range(NBUF)]
            + [pltpu.SemaphoreType.DMA for _ in range(2 * NBUF)]
        ),
    )


def kernel(input_ids, table):
    b, seq = input_ids.shape
    rows_per_w = b // NW
    assert rows_per_w * NW == b and rows_per_w % NBUF == 0
    assert seq % 40 == 0 and seq // 5 <= 128 and (seq * rows_per_w) % 8 == 0
    ids = input_ids.astype(jnp.int32)
    out = _make_gather(rows_per_w, seq)(ids, table)
    return out.reshape(b, seq, D)


# speedup vs baseline: 1.0119x; 1.0119x over previous
import functools

import jax
import jax.numpy as jnp
from jax import lax
from jax.experimental import pallas as pl
from jax.experimental.pallas import tpu as pltpu
from jax.experimental.pallas import tpu_sc as plsc

D = 128
CHUNK = 64
NBUF = 10

_info = plsc.get_sparse_core_info()
NC, NS = _info.num_cores, _info.num_subcores
NW = NC * NS


@functools.lru_cache(maxsize=None)
def _make_gather(n_chunks: int):
    mesh = plsc.VectorSubcoreMesh(core_axis_name="c", subcore_axis_name="s")
    n_blocks = n_chunks // NBUF
    total = NW * n_chunks * CHUNK

    def body(ids_hbm, table_hbm, out_hbm, idx_v, *rest):
        bufs = rest[:NBUF]
        gsems = rest[NBUF:2 * NBUF]
        wsems = rest[2 * NBUF:3 * NBUF]
        wid = lax.axis_index("s") * NC + lax.axis_index("c")
        row0 = wid * (n_chunks * CHUNK)

        pltpu.sync_copy(ids_hbm.at[wid], idx_v)

        def start_gather(g, b):
            pltpu.async_copy(table_hbm.at[idx_v.at[g]], bufs[b], gsems[b])

        def wait_gather(g, b):
            pltpu.make_async_copy(table_hbm.at[idx_v.at[g]], bufs[b],
                                  gsems[b]).wait()

        def start_write(g, b):
            pltpu.async_copy(bufs[b],
                             out_hbm.at[pl.ds(row0 + g * CHUNK, CHUNK)],
                             wsems[b])

        def wait_write(g, b):
            pltpu.make_async_copy(bufs[b],
                                  out_hbm.at[pl.ds(row0 + g * CHUNK, CHUNK)],
                                  wsems[b]).wait()

        for b in range(NBUF):
            start_gather(b, b)

        def block(i, carry):
            base = i * NBUF
            for b in range(NBUF):
                wait_gather(base + b, b)
                start_write(base + b, b)
            for b in range(NBUF):
                wait_write(base + b, b)
                start_gather(base + NBUF + b, b)
            return carry

        lax.fori_loop(0, n_blocks - 1, block, 0)

        base = (n_blocks - 1) * NBUF
        for b in range(NBUF):
            wait_gather(base + b, b)
            start_write(base + b, b)
        for b in range(NBUF):
            wait_write(base + b, b)

    return pl.kernel(
        body,
        out_type=jax.ShapeDtypeStruct((total, D), jnp.float32),
        mesh=mesh,
        scratch_types=(
            [pltpu.VMEM((n_chunks, CHUNK), jnp.int32)]
            + [pltpu.VMEM((CHUNK, D), jnp.float32) for _ in range(NBUF)]
            + [pltpu.SemaphoreType.DMA for _ in range(2 * NBUF)]
        ),
    )


def kernel(input_ids, table):
    b, s = input_ids.shape
    total = b * s
    n_chunks = total // (NW * CHUNK)
    assert n_chunks * NW * CHUNK == total and n_chunks % NBUF == 0
    ids3d = input_ids.reshape(NW, n_chunks, CHUNK).astype(jnp.int32)
    out = _make_gather(n_chunks)(ids3d, table)
    return out.reshape(b, s, D)
